# baseline (speedup 1.0000x reference)
import functools

import jax
import jax.numpy as jnp
from jax import lax
from jax.experimental import pallas as pl
from jax.experimental.pallas import tpu as pltpu
from jax.experimental.pallas import tpu_sc as plsc

N_NODES = 10000
N_PAD = 10240
N_EDGES = 320000
D_FEAT = 128
D_OUT = 16

NTILE = 16
E_PAD = 327680
CH = 128
NCH = 80
HALVES1 = 2
RPT = N_PAD // NTILE

_SC_PARAMS = pltpu.CompilerParams(use_tc_tiling_on_sc=False)


def _mm1_body(x_ref, w_ref, o_ref):
    o_ref[...] = jnp.dot(x_ref[...], w_ref[...],
                         preferred_element_type=jnp.float32)


def _matmul1(x, w1):
    bm = 1000
    return pl.pallas_call(
        _mm1_body,
        grid=(N_NODES // bm,),
        in_specs=[
            pl.BlockSpec((bm, D_FEAT), lambda i: (i, 0)),
            pl.BlockSpec((D_FEAT, D_FEAT), lambda i: (0, 0)),
        ],
        out_specs=pl.BlockSpec((bm, D_FEAT), lambda i: (i, 0)),
        out_shape=jax.ShapeDtypeStruct((N_NODES, D_FEAT), jnp.float32),
    )(x, w1)


def _mm2_body(p0_ref, p1_ref, w_ref, o_ref):
    h = jnp.maximum(p0_ref[...] + p1_ref[...], 0.0)
    o_ref[...] = jnp.dot(h, w_ref[...], preferred_element_type=jnp.float32)


def _combine_mm2(p0, p1, w2):
    bm = 1000
    return pl.pallas_call(
        _mm2_body,
        grid=(N_NODES // bm,),
        in_specs=[
            pl.BlockSpec((bm, D_FEAT), lambda i: (i, 0)),
            pl.BlockSpec((bm, D_FEAT), lambda i: (i, 0)),
            pl.BlockSpec((D_FEAT, D_OUT), lambda i: (0, 0)),
        ],
        out_specs=pl.BlockSpec((bm, D_OUT), lambda i: (i, 0)),
        out_shape=jax.ShapeDtypeStruct((N_NODES, D_OUT), jnp.float32),
    )(p0, p1, w2)


def _add_body(a_ref, b_ref, o_ref):
    o_ref[...] = a_ref[...] + b_ref[...]


def _final_add(q0, q1):
    bm = 2000
    return pl.pallas_call(
        _add_body,
        grid=(N_NODES // bm,),
        in_specs=[
            pl.BlockSpec((bm, D_OUT), lambda i: (i, 0)),
            pl.BlockSpec((bm, D_OUT), lambda i: (i, 0)),
        ],
        out_specs=pl.BlockSpec((bm, D_OUT), lambda i: (i, 0)),
        out_shape=jax.ShapeDtypeStruct((N_NODES, D_OUT), jnp.float32),
    )(q0, q1)


def _scale_rows(dst, src, wv, c, d, n_edges):
    for q in range(n_edges // 16):
        wvec = wv[c, pl.ds(q * 16, 16)]
        for j in range(16):
            e = q * 16 + j
            ws = wvec[j]
            for g in range(d // 16):
                sl = pl.ds(g * 16, 16)
                dst[e, sl] = src[e, sl] * ws


_MESH = plsc.VectorSubcoreMesh(core_axis_name="c", subcore_axis_name="s")


def _make_spmm1():
    d = D_FEAT
    nbuf = NCH // HALVES1
    npair = nbuf // 2

    @functools.partial(
        pl.kernel,
        out_type=jax.ShapeDtypeStruct((2, N_PAD, d), jnp.float32),
        mesh=_MESH,
        compiler_params=_SC_PARAMS,
        scratch_types=[
            pltpu.VMEM((nbuf, CH), jnp.int32),
            pltpu.VMEM((nbuf, CH), jnp.int32),
            pltpu.VMEM((nbuf, CH), jnp.float32),
            pltpu.VMEM((CH, d), jnp.float32),
            pltpu.VMEM((CH, d), jnp.float32),
            pltpu.VMEM_SHARED((N_PAD, d), jnp.float32),
            pltpu.SemaphoreType.DMA,
            pltpu.SemaphoreType.DMA,
        ],
    )
    def spmm(pre_hbm, src_hbm, dst_hbm, w_hbm, zero_hbm, out_hbm,
             srcv, dstv, wv, b0, b1, acc, gsem0, gsem1):
        cid = lax.axis_index("c")
        sid = lax.axis_index("s")

        pltpu.sync_copy(zero_hbm.at[pl.ds(sid * RPT, RPT)],
                        acc.at[pl.ds(sid * RPT, RPT)])
        plsc.subcore_barrier()

        def half(c, buf, gsem):
            pltpu.make_async_copy(pre_hbm.at[srcv.at[c]], buf, gsem).wait()
            _scale_rows(buf, buf, wv, c, d, CH)
            pltpu.sync_copy(buf, acc.at[dstv.at[c]], add=True)

            @pl.when(c < nbuf - 2)
            def _():
                pltpu.async_copy(pre_hbm.at[srcv.at[c + 2]], buf, gsem)

        def pair(i, carry):
            half(2 * i, b0, gsem0)
            half(2 * i + 1, b1, gsem1)
            return carry

        def stage(hv, carry):
            pltpu.sync_copy(src_hbm.at[cid, sid, pl.ds(hv * nbuf, nbuf)],
                            srcv)
            pltpu.sync_copy(dst_hbm.at[cid, sid, pl.ds(hv * nbuf, nbuf)],
                            dstv)
            pltpu.sync_copy(w_hbm.at[cid, sid, pl.ds(hv * nbuf, nbuf)], wv)
            pltpu.async_copy(pre_hbm.at[srcv.at[0]], b0, gsem0)
            pltpu.async_copy(pre_hbm.at[srcv.at[1]], b1, gsem1)
            lax.fori_loop(0, npair, pair, 0)
            return carry

        lax.fori_loop(0, HALVES1, stage, 0)
        plsc.subcore_barrier()

        pltpu.sync_copy(acc.at[pl.ds(sid * RPT, RPT)],
                        out_hbm.at[cid, pl.ds(sid * RPT, RPT)])

    return spmm


def _make_spmm2():
    d = D_OUT
    npair = NCH // 2

    @functools.partial(
        pl.kernel,
        out_type=jax.ShapeDtypeStruct((2, N_PAD, d), jnp.float32),
        mesh=_MESH,
        compiler_params=_SC_PARAMS,
        scratch_types=[
            pltpu.VMEM((NCH, CH), jnp.int32),
            pltpu.VMEM((NCH, CH), jnp.int32),
            pltpu.VMEM((NCH, CH), jnp.float32),
            pltpu.VMEM((CH, d), jnp.float32),
            pltpu.VMEM((CH, d), jnp.float32),
            pltpu.VMEM((CH, d), jnp.float32),
            pltpu.VMEM((CH, d), jnp.float32),
            pltpu.VMEM_SHARED((N_PAD, d), jnp.float32),
            pltpu.SemaphoreType.DMA,
            pltpu.SemaphoreType.DMA,
            pltpu.SemaphoreType.DMA,
            pltpu.SemaphoreType.DMA,
        ],
    )
    def spmm(pre_hbm, src_hbm, dst_hbm, w_hbm, zero_hbm, out_hbm,
             srcv, dstv, wv, g0, g1, s0, s1, acc,
             gsem0, gsem1, ssem0, ssem1):
        cid = lax.axis_index("c")
        sid = lax.axis_index("s")

        pltpu.sync_copy(zero_hbm.at[pl.ds(sid * RPT, RPT)],
                        acc.at[pl.ds(sid * RPT, RPT)])
        pltpu.sync_copy(src_hbm.at[cid, sid], srcv)
        pltpu.sync_copy(dst_hbm.at[cid, sid], dstv)
        pltpu.sync_copy(w_hbm.at[cid, sid], wv)
        plsc.subcore_barrier()

        pltpu.async_copy(pre_hbm.at[srcv.at[0]], g0, gsem0)
        pltpu.async_copy(pre_hbm.at[srcv.at[1]], g1, gsem1)

        def half(i, c, gbuf, sbuf, gsem, ssem):
            pltpu.make_async_copy(pre_hbm.at[srcv.at[c]], gbuf, gsem).wait()

            @pl.when(i > 0)
            def _():
                pltpu.make_async_copy(
                    sbuf, acc.at[dstv.at[c - 2]], ssem).wait()

            _scale_rows(sbuf, gbuf, wv, c, d, CH)
            pltpu.async_copy(sbuf, acc.at[dstv.at[c]], ssem, add=True)

            @pl.when(i < npair - 1)
            def _():
                pltpu.async_copy(pre_hbm.at[srcv.at[c + 2]], gbuf, gsem)

        def pair(i, carry):
            half(i, 2 * i, g0, s0, gsem0, ssem0)
            half(i, 2 * i + 1, g1, s1, gsem1, ssem1)
            return carry

        lax.fori_loop(0, npair, pair, 0)
        pltpu.make_async_copy(s0, acc.at[dstv.at[NCH - 2]], ssem0).wait()
        pltpu.make_async_copy(s1, acc.at[dstv.at[NCH - 1]], ssem1).wait()
        plsc.subcore_barrier()

        pltpu.sync_copy(acc.at[pl.ds(sid * RPT, RPT)],
                        out_hbm.at[cid, pl.ds(sid * RPT, RPT)])

    return spmm


_spmm1 = _make_spmm1()
_spmm2 = _make_spmm2()


def kernel(x, edge_index, edge_weight, W1, W2):
    src = edge_index[0].astype(jnp.int32)
    dst = edge_index[1].astype(jnp.int32)
    ew = edge_weight.astype(jnp.float32)

    pad = E_PAD - N_EDGES
    src = jnp.concatenate([src, jnp.zeros((pad,), jnp.int32)])
    dst = jnp.concatenate([dst, jnp.zeros((pad,), jnp.int32)])
    ew = jnp.concatenate([ew, jnp.zeros((pad,), jnp.float32)])
    srcr = src.reshape(2, NTILE, NCH, CH)
    dstr = dst.reshape(2, NTILE, NCH, CH)
    ewr = ew.reshape(2, NTILE, NCH, CH)

    zero128 = jnp.zeros((N_PAD, D_FEAT), jnp.float32)
    zero16 = jnp.zeros((N_PAD, D_OUT), jnp.float32)

    pre1 = _matmul1(x, W1)
    parts1 = _spmm1(pre1, srcr, dstr, ewr, zero128)
    pre2 = _combine_mm2(parts1[0], parts1[1], W2)
    parts2 = _spmm2(pre2, srcr, dstr, ewr, zero16)
    return _final_add(parts2[0], parts2[1])


# speedup vs baseline: 1.2194x; 1.2194x over previous
import functools

import jax
import jax.numpy as jnp
from jax import lax
from jax.experimental import pallas as pl
from jax.experimental.pallas import tpu as pltpu
from jax.experimental.pallas import tpu_sc as plsc

N_NODES = 10000
N_PAD = 10240
N_EDGES = 320000
D_FEAT = 128
D_HALF = 64
D_OUT = 16

NTILE = 16
E_PAD = 327680
CH1 = 128
NCH1 = 160
HALVES1 = 2
CH2 = 128
NCH2 = 80
RPT = N_PAD // NTILE

_SC_PARAMS = pltpu.CompilerParams(use_tc_tiling_on_sc=False)


def _mm1_body(x_ref, w_ref, o_ref):
    res = jnp.dot(x_ref[...], w_ref[...], preferred_element_type=jnp.float32)
    o_ref[0] = res[:, :D_HALF]
    o_ref[1] = res[:, D_HALF:]


def _matmul1(x, w1):
    bm = 1000
    return pl.pallas_call(
        _mm1_body,
        grid=(N_NODES // bm,),
        in_specs=[
            pl.BlockSpec((bm, D_FEAT), lambda i: (i, 0)),
            pl.BlockSpec((D_FEAT, D_FEAT), lambda i: (0, 0)),
        ],
        out_specs=pl.BlockSpec((2, bm, D_HALF), lambda i: (0, i, 0)),
        out_shape=jax.ShapeDtypeStruct((2, N_NODES, D_HALF), jnp.float32),
    )(x, w1)


def _mm2_body(h0_ref, h1_ref, wa_ref, wb_ref, o_ref):
    a = jnp.maximum(h0_ref[...], 0.0)
    b = jnp.maximum(h1_ref[...], 0.0)
    o_ref[...] = (jnp.dot(a, wa_ref[...], preferred_element_type=jnp.float32)
                  + jnp.dot(b, wb_ref[...], preferred_element_type=jnp.float32))


def _combine_mm2(h0, h1, w2a, w2b):
    bm = 1000
    return pl.pallas_call(
        _mm2_body,
        grid=(N_NODES // bm,),
        in_specs=[
            pl.BlockSpec((bm, D_HALF), lambda i: (i, 0)),
            pl.BlockSpec((bm, D_HALF), lambda i: (i, 0)),
            pl.BlockSpec((D_HALF, D_OUT), lambda i: (0, 0)),
            pl.BlockSpec((D_HALF, D_OUT), lambda i: (0, 0)),
        ],
        out_specs=pl.BlockSpec((bm, D_OUT), lambda i: (i, 0)),
        out_shape=jax.ShapeDtypeStruct((N_NODES, D_OUT), jnp.float32),
    )(h0, h1, w2a, w2b)


def _add_body(a_ref, b_ref, o_ref):
    o_ref[...] = a_ref[...] + b_ref[...]


def _final_add(q0, q1):
    bm = 2000
    return pl.pallas_call(
        _add_body,
        grid=(N_NODES // bm,),
        in_specs=[
            pl.BlockSpec((bm, D_OUT), lambda i: (i, 0)),
            pl.BlockSpec((bm, D_OUT), lambda i: (i, 0)),
        ],
        out_specs=pl.BlockSpec((bm, D_OUT), lambda i: (i, 0)),
        out_shape=jax.ShapeDtypeStruct((N_NODES, D_OUT), jnp.float32),
    )(q0, q1)


def _scale_rows(dst, src, wv, c, d, n_edges):
    for q in range(n_edges // 16):
        wvec = wv[c, pl.ds(q * 16, 16)]
        for j in range(16):
            e = q * 16 + j
            ws = wvec[j]
            for g in range(d // 16):
                sl = pl.ds(g * 16, 16)
                dst[e, sl] = src[e, sl] * ws


_MESH = plsc.VectorSubcoreMesh(core_axis_name="c", subcore_axis_name="s")


def _make_spmm(d, nch, ch, halves=1, core_offset=0):
    nbuf = nch // halves
    npair = nbuf // 2

    @functools.partial(
        pl.kernel,
        out_type=jax.ShapeDtypeStruct((2, N_PAD, d), jnp.float32),
        mesh=_MESH,
        compiler_params=_SC_PARAMS,
        scratch_types=[
            pltpu.VMEM((nbuf, ch), jnp.int32),
            pltpu.VMEM((nbuf, ch), jnp.int32),
            pltpu.VMEM((nbuf, ch), jnp.float32),
            pltpu.VMEM((ch, d), jnp.float32),
            pltpu.VMEM((ch, d), jnp.float32),
            pltpu.VMEM((ch, d), jnp.float32),
            pltpu.VMEM((ch, d), jnp.float32),
            pltpu.VMEM_SHARED((N_PAD, d), jnp.float32),
            pltpu.SemaphoreType.DMA,
            pltpu.SemaphoreType.DMA,
            pltpu.SemaphoreType.DMA,
            pltpu.SemaphoreType.DMA,
        ],
    )
    def spmm(pre_hbm, src_hbm, dst_hbm, w_hbm, zero_hbm, out_hbm,
             srcv, dstv, wv, g0, g1, s0, s1, acc,
             gsem0, gsem1, ssem0, ssem1):
        cid = lax.axis_index("c")
        sid = lax.axis_index("s")

        pltpu.sync_copy(zero_hbm.at[pl.ds(sid * RPT, RPT)],
                        acc.at[pl.ds(sid * RPT, RPT)])
        plsc.subcore_barrier()

        def half(i, c, gbuf, sbuf, gsem, ssem):
            pltpu.make_async_copy(pre_hbm.at[srcv.at[c]], gbuf, gsem).wait()

            @pl.when(i > 0)
            def _():
                pltpu.make_async_copy(
                    sbuf, acc.at[dstv.at[c - 2]], ssem).wait()

            _scale_rows(sbuf, gbuf, wv, c, d, ch)
            pltpu.async_copy(sbuf, acc.at[dstv.at[c]], ssem, add=True)

            @pl.when(i < npair - 1)
            def _():
                pltpu.async_copy(pre_hbm.at[srcv.at[c + 2]], gbuf, gsem)

        def pair(i, carry):
            half(i, 2 * i, g0, s0, gsem0, ssem0)
            half(i, 2 * i + 1, g1, s1, gsem1, ssem1)
            return carry

        def stage(hv, carry):
            if core_offset:
                pltpu.sync_copy(src_hbm.at[sid, pl.ds(hv * nbuf, nbuf)],
                                srcv)
                pltpu.sync_copy(dst_hbm.at[sid, pl.ds(hv * nbuf, nbuf)],
                                dstv)
                pltpu.sync_copy(w_hbm.at[sid, pl.ds(hv * nbuf, nbuf)], wv)
                off = cid * core_offset
                for r in range(nbuf):
                    for g in range(ch // 16):
                        sl = pl.ds(g * 16, 16)
                        srcv[r, sl] = srcv[r, sl] + off
            else:
                pltpu.sync_copy(src_hbm.at[cid, sid, pl.ds(hv * nbuf, nbuf)],
                                srcv)
                pltpu.sync_copy(dst_hbm.at[cid, sid, pl.ds(hv * nbuf, nbuf)],
                                dstv)
                pltpu.sync_copy(w_hbm.at[cid, sid, pl.ds(hv * nbuf, nbuf)],
                                wv)
            pltpu.async_copy(pre_hbm.at[srcv.at[0]], g0, gsem0)
            pltpu.async_copy(pre_hbm.at[srcv.at[1]], g1, gsem1)
            lax.fori_loop(0, npair, pair, 0)
            pltpu.make_async_copy(s0, acc.at[dstv.at[nbuf - 2]], ssem0).wait()
            pltpu.make_async_copy(s1, acc.at[dstv.at[nbuf - 1]], ssem1).wait()
            return carry

        lax.fori_loop(0, halves, stage, 0)
        plsc.subcore_barrier()

        pltpu.sync_copy(acc.at[pl.ds(sid * RPT, RPT)],
                        out_hbm.at[cid, pl.ds(sid * RPT, RPT)])

    return spmm


_spmm1 = _make_spmm(D_HALF, NCH1, CH1, HALVES1, core_offset=N_NODES)
_spmm2 = _make_spmm(D_OUT, NCH2, CH2)


def kernel(x, edge_index, edge_weight, W1, W2):
    src = edge_index[0].astype(jnp.int32)
    dst = edge_index[1].astype(jnp.int32)
    ew = edge_weight.astype(jnp.float32)

    pad = E_PAD - N_EDGES
    src = jnp.concatenate([src, jnp.zeros((pad,), jnp.int32)])
    dst = jnp.concatenate([dst, jnp.zeros((pad,), jnp.int32)])
    ew = jnp.concatenate([ew, jnp.zeros((pad,), jnp.float32)])

    src1 = src.reshape(NTILE, NCH1, CH1)
    dst1 = dst.reshape(NTILE, NCH1, CH1)
    ew1 = ew.reshape(NTILE, NCH1, CH1)
    src2 = src.reshape(2, NTILE, NCH2, CH2)
    dst2 = dst.reshape(2, NTILE, NCH2, CH2)
    ew2 = ew.reshape(2, NTILE, NCH2, CH2)

    zero64 = jnp.zeros((N_PAD, D_HALF), jnp.float32)
    zero16 = jnp.zeros((N_PAD, D_OUT), jnp.float32)

    pre1 = _matmul1(x, W1).reshape(2 * N_NODES, D_HALF)
    h = _spmm1(pre1, src1, dst1, ew1, zero64)
    pre2 = _combine_mm2(h[0], h[1], W2[:D_HALF], W2[D_HALF:])
    parts2 = _spmm2(pre2, src2, dst2, ew2, zero16)
    return _final_add(parts2[0], parts2[1])


# speedup vs baseline: 1.3061x; 1.0711x over previous
import functools

import jax
import jax.numpy as jnp
from jax import lax
from jax.experimental import pallas as pl
from jax.experimental.pallas import tpu as pltpu
from jax.experimental.pallas import tpu_sc as plsc

N_NODES = 10000
N_PAD = 10240
N_EDGES = 320000
D_FEAT = 128
D_HALF = 64
D_OUT = 16

NTILE = 16
E_PAD = 327680
CH1 = 128
NCH1 = 160
HALVES1 = 2
CH2 = 128
NCH2 = 80
RPT = N_PAD // NTILE

_SC_PARAMS = pltpu.CompilerParams(use_tc_tiling_on_sc=False)


def _mm1_body(x_ref, w_ref, o_ref):
    res = jnp.dot(x_ref[...], w_ref[...], preferred_element_type=jnp.float32)
    o_ref[0] = res[:, :D_HALF]
    o_ref[1] = res[:, D_HALF:]


def _matmul1(x, w1):
    bm = 1000
    return pl.pallas_call(
        _mm1_body,
        grid=(N_NODES // bm,),
        in_specs=[
            pl.BlockSpec((bm, D_FEAT), lambda i: (i, 0)),
            pl.BlockSpec((D_FEAT, D_FEAT), lambda i: (0, 0)),
        ],
        out_specs=pl.BlockSpec((2, bm, D_HALF), lambda i: (0, i, 0)),
        out_shape=jax.ShapeDtypeStruct((2, N_NODES, D_HALF), jnp.float32),
    )(x, w1)


def _mm2_body(h0_ref, h1_ref, wa_ref, wb_ref, o_ref):
    a = jnp.maximum(h0_ref[...], 0.0)
    b = jnp.maximum(h1_ref[...], 0.0)
    o_ref[...] = (jnp.dot(a, wa_ref[...], preferred_element_type=jnp.float32)
                  + jnp.dot(b, wb_ref[...], preferred_element_type=jnp.float32))


def _combine_mm2(h0, h1, w2a, w2b):
    bm = 1000
    return pl.pallas_call(
        _mm2_body,
        grid=(N_NODES // bm,),
        in_specs=[
            pl.BlockSpec((bm, D_HALF), lambda i: (i, 0)),
            pl.BlockSpec((bm, D_HALF), lambda i: (i, 0)),
            pl.BlockSpec((D_HALF, D_OUT), lambda i: (0, 0)),
            pl.BlockSpec((D_HALF, D_OUT), lambda i: (0, 0)),
        ],
        out_specs=pl.BlockSpec((bm, D_OUT), lambda i: (i, 0)),
        out_shape=jax.ShapeDtypeStruct((N_NODES, D_OUT), jnp.float32),
    )(h0, h1, w2a, w2b)


def _add_body(a_ref, b_ref, o_ref):
    o_ref[...] = a_ref[...] + b_ref[...]


def _final_add(q0, q1):
    bm = 2000
    return pl.pallas_call(
        _add_body,
        grid=(N_NODES // bm,),
        in_specs=[
            pl.BlockSpec((bm, D_OUT), lambda i: (i, 0)),
            pl.BlockSpec((bm, D_OUT), lambda i: (i, 0)),
        ],
        out_specs=pl.BlockSpec((bm, D_OUT), lambda i: (i, 0)),
        out_shape=jax.ShapeDtypeStruct((N_NODES, D_OUT), jnp.float32),
    )(q0, q1)


def _scale_rows(dst, src, wv, c, d, n_edges):
    for q in range(n_edges // 16):
        wvec = wv[c, pl.ds(q * 16, 16)]
        for j in range(16):
            e = q * 16 + j
            ws = wvec[j]
            for g in range(d // 16):
                sl = pl.ds(g * 16, 16)
                dst[e, sl] = src[e, sl] * ws


_MESH = plsc.VectorSubcoreMesh(core_axis_name="c", subcore_axis_name="s")


def _make_spmm(d, nch, ch, halves=1):
    nbuf = nch // halves
    npair = nbuf // 2

    @functools.partial(
        pl.kernel,
        out_type=jax.ShapeDtypeStruct((2, N_PAD, d), jnp.float32),
        mesh=_MESH,
        compiler_params=_SC_PARAMS,
        scratch_types=[
            pltpu.VMEM((nbuf, ch), jnp.int32),
            pltpu.VMEM((nbuf, ch), jnp.int32),
            pltpu.VMEM((nbuf, ch), jnp.float32),
            pltpu.VMEM((ch, d), jnp.float32),
            pltpu.VMEM((ch, d), jnp.float32),
            pltpu.VMEM((ch, d), jnp.float32),
            pltpu.VMEM((ch, d), jnp.float32),
            pltpu.VMEM_SHARED((N_PAD, d), jnp.float32),
            pltpu.SemaphoreType.DMA,
            pltpu.SemaphoreType.DMA,
            pltpu.SemaphoreType.DMA,
            pltpu.SemaphoreType.DMA,
        ],
    )
    def spmm(pre_hbm, src_hbm, dst_hbm, w_hbm, zero_hbm, out_hbm,
             srcv, dstv, wv, g0, g1, s0, s1, acc,
             gsem0, gsem1, ssem0, ssem1):
        cid = lax.axis_index("c")
        sid = lax.axis_index("s")

        pltpu.sync_copy(zero_hbm.at[pl.ds(sid * RPT, RPT)],
                        acc.at[pl.ds(sid * RPT, RPT)])
        plsc.subcore_barrier()

        def half(i, c, gbuf, sbuf, gsem, ssem):
            pltpu.make_async_copy(pre_hbm.at[srcv.at[c]], gbuf, gsem).wait()

            @pl.when(i > 0)
            def _():
                pltpu.make_async_copy(
                    sbuf, acc.at[dstv.at[c - 2]], ssem).wait()

            _scale_rows(sbuf, gbuf, wv, c, d, ch)
            pltpu.async_copy(sbuf, acc.at[dstv.at[c]], ssem, add=True)

            @pl.when(i < npair - 1)
            def _():
                pltpu.async_copy(pre_hbm.at[srcv.at[c + 2]], gbuf, gsem)

        def pair(i, carry):
            half(i, 2 * i, g0, s0, gsem0, ssem0)
            half(i, 2 * i + 1, g1, s1, gsem1, ssem1)
            return carry

        def stage(hv, carry):
            pltpu.sync_copy(src_hbm.at[cid, sid, pl.ds(hv * nbuf, nbuf)],
                            srcv)
            pltpu.sync_copy(dst_hbm.at[cid, sid, pl.ds(hv * nbuf, nbuf)],
                            dstv)
            pltpu.sync_copy(w_hbm.at[cid, sid, pl.ds(hv * nbuf, nbuf)], wv)
            pltpu.async_copy(pre_hbm.at[srcv.at[0]], g0, gsem0)
            pltpu.async_copy(pre_hbm.at[srcv.at[1]], g1, gsem1)
            lax.fori_loop(0, npair, pair, 0)
            pltpu.make_async_copy(s0, acc.at[dstv.at[nbuf - 2]], ssem0).wait()
            pltpu.make_async_copy(s1, acc.at[dstv.at[nbuf - 1]], ssem1).wait()
            return carry

        lax.fori_loop(0, halves, stage, 0)
        plsc.subcore_barrier()

        pltpu.sync_copy(acc.at[pl.ds(sid * RPT, RPT)],
                        out_hbm.at[cid, pl.ds(sid * RPT, RPT)])

    return spmm


_spmm1 = _make_spmm(D_HALF, NCH1, CH1, HALVES1)
_spmm2 = _make_spmm(D_OUT, NCH2, CH2)


def kernel(x, edge_index, edge_weight, W1, W2):
    src = edge_index[0].astype(jnp.int32)
    dst = edge_index[1].astype(jnp.int32)
    ew = edge_weight.astype(jnp.float32)

    pad = E_PAD - N_EDGES
    src = jnp.concatenate([src, jnp.zeros((pad,), jnp.int32)])
    dst = jnp.concatenate([dst, jnp.zeros((pad,), jnp.int32)])
    ew = jnp.concatenate([ew, jnp.zeros((pad,), jnp.float32)])

    src1 = jnp.stack([src, src + N_NODES]).reshape(2, NTILE, NCH1, CH1)
    dst1 = jnp.broadcast_to(dst.reshape(1, NTILE, NCH1, CH1),
                            (2, NTILE, NCH1, CH1))
    ew1 = jnp.broadcast_to(ew.reshape(1, NTILE, NCH1, CH1),
                           (2, NTILE, NCH1, CH1))
    src2 = src.reshape(2, NTILE, NCH2, CH2)
    dst2 = dst.reshape(2, NTILE, NCH2, CH2)
    ew2 = ew.reshape(2, NTILE, NCH2, CH2)

    zero64 = jnp.zeros((N_PAD, D_HALF), jnp.float32)
    zero16 = jnp.zeros((N_PAD, D_OUT), jnp.float32)

    pre1 = _matmul1(x, W1).reshape(2 * N_NODES, D_HALF)
    h = _spmm1(pre1, src1, dst1, ew1, zero64)
    pre2 = _combine_mm2(h[0], h[1], W2[:D_HALF], W2[D_HALF:])
    parts2 = _spmm2(pre2, src2, dst2, ew2, zero16)
    return _final_add(parts2[0], parts2[1])
